# Initial kernel scaffold; baseline (speedup 1.0000x reference)
import jax
import jax.numpy as jnp
from jax.experimental import pallas as pl


def kernel(vectors, x, V, params, senders, species):
    raise NotImplementedError("write your pallas kernel here")


# speedup vs baseline: 1.5179x; 1.5179x over previous
import functools

import jax
import jax.numpy as jnp
from jax import lax
from jax.experimental import pallas as pl
from jax.experimental.pallas import tpu as pltpu
from jax.experimental.pallas import tpu_sc as plsc

f32 = jnp.float32
bf16 = jnp.bfloat16

N_EDGES = 160000
N_NODES = 10000
NP = 10240
D_FEAT = 256
CE = 64
HID = 512

BLK = 1280
NB = N_EDGES // BLK

SC_ROWS = 1280
SC_COLS = 125
SC_TILES = 32
SC_RPT = SC_ROWS // SC_TILES

_EA = -28.0
_EB = 48.0
_EC = -21.0


def _chi_env_body(x_ref, v_ref, w1_ref, w2_ref, w3_ref, chi_ref, env_ref):
    h = jnp.dot(x_ref[:].astype(bf16), w1_ref[:], preferred_element_type=f32)
    h = h * jax.nn.sigmoid(h)
    h = jnp.dot(h.astype(bf16), w2_ref[:], preferred_element_type=f32)
    h = h * jax.nn.sigmoid(h)
    chi_ref[:] = jnp.sum(h * w3_ref[:], axis=1, keepdims=True)
    v = v_ref[:]
    d = jnp.sqrt(jnp.sum(v * v, axis=1, keepdims=True))
    u = 1.0 + _EA * d**6 + _EB * d**7 + _EC * d**8
    env_ref[:] = jnp.where(d < 1.0, u, 0.0)


def _node_body(p_ref, sp_ref, tab_ref, w0a_ref, w0b_ref, w1w_ref, sc_ref,
               q_ref, w_ref, pot_ref):
    chi_scale = sc_ref[0, 0]
    gamma_scale = sc_ref[0, 1]
    gamma_shift = sc_ref[0, 2]
    chis = (p_ref[:, 0:1] + p_ref[:, 1:2]) * chi_scale
    ks = lax.broadcasted_iota(jnp.int32, (NP, 128), 1)
    oh = (sp_ref[:] == ks).astype(f32)
    g = jnp.dot(oh, tab_ref[:], preferred_element_type=f32,
                precision=jax.lax.Precision.HIGHEST)
    ce = g[:, 0:CE]
    gam = g[:, CE:CE + 1] * gamma_scale + gamma_shift
    hraw = g[:, CE + 1:CE + 2]
    soft = jnp.maximum(hraw, 0.0) + jnp.log(1.0 + jnp.exp(-jnp.abs(hraw)))
    h_eff = soft + 1.0 / gam
    q = -chis / h_eff
    q_ref[:] = q
    pot_ref[:, :] = jnp.sum(0.5 * h_eff * q * q + chis * q).reshape(1, 1)
    h = q * w0a_ref[:] + jnp.dot(ce, w0b_ref[:], preferred_element_type=f32,
                                 precision=jax.lax.Precision.HIGHEST)
    h = h * jax.nn.sigmoid(h)
    w_ref[:] = jnp.dot(h, w1w_ref[:], preferred_element_type=f32,
                       precision=jax.lax.Precision.HIGHEST)


def _edge_mlp_body(x_ref, we_ref, env_ref, w1a_ref, w1b_ref, w2_ref, w3_ref,
                   out_ref):
    h = jnp.dot(x_ref[:].astype(bf16), w1a_ref[:], preferred_element_type=f32)
    h = h + jnp.dot(we_ref[:].astype(bf16), w1b_ref[:],
                    preferred_element_type=f32)
    h = h * jax.nn.sigmoid(h)
    h = jnp.dot(h.astype(bf16), w2_ref[:], preferred_element_type=f32)
    h = h * jax.nn.sigmoid(h)
    h = jnp.dot(h.astype(bf16), w3_ref[:], preferred_element_type=f32)
    out_ref[:] = env_ref[:] * h


SC_EPT = N_EDGES // SC_TILES
SC_GCHUNK = 128
SC_GFULL = SC_EPT // SC_GCHUNK
SC_GTAIL = SC_EPT - SC_GFULL * SC_GCHUNK


def _sc_scatter_add(chis2d, snd2d, zeros_np):
    mesh = plsc.VectorSubcoreMesh(core_axis_name="c", subcore_axis_name="s")

    @functools.partial(
        pl.kernel,
        out_type=jax.ShapeDtypeStruct((2 * NP,), f32),
        mesh=mesh,
        scratch_types=[
            pltpu.VMEM((SC_RPT, SC_COLS), jnp.int32),
            pltpu.VMEM((SC_RPT, SC_COLS), f32),
            pltpu.VMEM_SHARED((NP,), f32),
        ],
    )
    def _scatter_k(chis_hbm, snd_hbm, zero_hbm, out_hbm, idx_v, val_v, acc_sh):
        c = lax.axis_index("c")
        s = lax.axis_index("s")
        wid = c * 16 + s

        @pl.when(s == 0)
        def _():
            pltpu.sync_copy(zero_hbm, acc_sh)

        plsc.subcore_barrier()
        pltpu.sync_copy(snd_hbm.at[pl.ds(wid * SC_RPT, SC_RPT)], idx_v)
        pltpu.sync_copy(chis_hbm.at[pl.ds(wid * SC_RPT, SC_RPT)], val_v)

        def body(j, carry):
            pltpu.sync_copy(val_v.at[j], acc_sh.at[idx_v.at[j]], add=True)
            return carry

        lax.fori_loop(0, SC_RPT, body, 0)
        plsc.subcore_barrier()

        @pl.when(s == 0)
        def _():
            pltpu.sync_copy(acc_sh, out_hbm.at[pl.ds(c * NP, NP)])

    return _scatter_k(chis2d, snd2d, zeros_np)


def _sc_gather_rows(w_nodes, snd_flat):
    mesh = plsc.VectorSubcoreMesh(core_axis_name="c", subcore_axis_name="s")

    @functools.partial(
        pl.kernel,
        out_type=jax.ShapeDtypeStruct((N_EDGES, 128), f32),
        mesh=mesh,
        scratch_types=[
            pltpu.VMEM((SC_EPT,), jnp.int32),
            pltpu.VMEM((SC_GCHUNK, 128), f32),
            pltpu.VMEM((SC_GTAIL, 128), f32),
            pltpu.SemaphoreType.DMA,
        ],
    )
    def _gather_k(w_hbm, snd_hbm, out_hbm, idx_v, rows_v, tail_v, sem):
        c = lax.axis_index("c")
        s = lax.axis_index("s")
        wid = c * 16 + s
        base = wid * SC_EPT
        pltpu.sync_copy(snd_hbm.at[pl.ds(base, SC_EPT)], idx_v)

        def body(j, carry):
            pltpu.async_copy(
                w_hbm.at[idx_v.at[pl.ds(j * SC_GCHUNK, SC_GCHUNK)]],
                rows_v, sem).wait()
            pltpu.sync_copy(
                rows_v, out_hbm.at[pl.ds(base + j * SC_GCHUNK, SC_GCHUNK)])
            return carry

        lax.fori_loop(0, SC_GFULL, body, 0)
        pltpu.async_copy(
            w_hbm.at[idx_v.at[pl.ds(SC_GFULL * SC_GCHUNK, SC_GTAIL)]],
            tail_v, sem).wait()
        pltpu.sync_copy(
            tail_v,
            out_hbm.at[pl.ds(base + SC_GFULL * SC_GCHUNK, SC_GTAIL)])

    return _gather_k(w_nodes, snd_flat)


def kernel(vectors, x, V, params, senders, species):
    p = params

    w1c = (p['W_chi'][0] / jnp.sqrt(jnp.float32(D_FEAT))).astype(bf16)
    w2c = (p['W_chi'][1] / jnp.sqrt(jnp.float32(CE))).astype(bf16)
    w3c = (p['W_chi'][2][:, 0] / jnp.sqrt(jnp.float32(CE))).reshape(1, CE)

    chi_e, env_e = pl.pallas_call(
        _chi_env_body,
        grid=(NB,),
        in_specs=[
            pl.BlockSpec((BLK, D_FEAT), lambda i: (i, 0)),
            pl.BlockSpec((BLK, 3), lambda i: (i, 0)),
            pl.BlockSpec((D_FEAT, CE), lambda i: (0, 0)),
            pl.BlockSpec((CE, CE), lambda i: (0, 0)),
            pl.BlockSpec((1, CE), lambda i: (0, 0)),
        ],
        out_specs=[
            pl.BlockSpec((BLK, 1), lambda i: (i, 0)),
            pl.BlockSpec((BLK, 1), lambda i: (i, 0)),
        ],
        out_shape=[
            jax.ShapeDtypeStruct((N_EDGES, 1), f32),
            jax.ShapeDtypeStruct((N_EDGES, 1), f32),
        ],
    )(x, vectors, w1c, w2c, w3c)

    chis2d = chi_e.reshape(SC_ROWS, SC_COLS)
    snd2d = senders.reshape(SC_ROWS, SC_COLS)
    partials = _sc_scatter_add(chis2d, snd2d,
                               jnp.zeros((NP,), f32)).reshape(2, NP)

    table = jnp.zeros((128, 128), f32)
    table = table.at[:100, :CE].set(p['charge_embed'])
    table = table.at[:100, CE].set(p['radius'])
    table = table.at[:100, CE + 1].set(p['hardness'])
    w0 = p['W_w'][0] / jnp.sqrt(jnp.float32(1 + CE))
    w0a = w0[0:1, :]
    w0b = w0[1:, :]
    w1w = jnp.zeros((CE, 128), f32).at[:, :CE].set(
        p['W_w'][1] / jnp.sqrt(jnp.float32(CE)))
    scal = jnp.stack([p['chi_scale'], p['gamma_scale'],
                      p['gamma_shift']]).reshape(1, 3)
    sp_col = jnp.pad(species, (0, NP - N_NODES)).reshape(NP, 1)

    q_pad, w_nodes, pot_arr = pl.pallas_call(
        _node_body,
        grid=(1,),
        in_specs=[
            pl.BlockSpec((NP, 2), lambda i: (0, 0)),
            pl.BlockSpec((NP, 1), lambda i: (0, 0)),
            pl.BlockSpec((128, 128), lambda i: (0, 0)),
            pl.BlockSpec((1, CE), lambda i: (0, 0)),
            pl.BlockSpec((CE, CE), lambda i: (0, 0)),
            pl.BlockSpec((CE, 128), lambda i: (0, 0)),
            pl.BlockSpec((1, 3), lambda i: (0, 0)),
        ],
        out_specs=[
            pl.BlockSpec((NP, 1), lambda i: (0, 0)),
            pl.BlockSpec((NP, 128), lambda i: (0, 0)),
            pl.BlockSpec((1, 1), lambda i: (0, 0)),
        ],
        out_shape=[
            jax.ShapeDtypeStruct((NP, 1), f32),
            jax.ShapeDtypeStruct((NP, 128), f32),
            jax.ShapeDtypeStruct((1, 1), f32),
        ],
    )(partials.T, sp_col, table, w0a, w0b, w1w, scal)

    w_edges = _sc_gather_rows(w_nodes, senders)

    wx0 = p['W_x'][0] / jnp.sqrt(jnp.float32(D_FEAT + CE))
    w1a = wx0[:D_FEAT].astype(bf16)
    w1b = jnp.zeros((128, HID), bf16).at[:CE].set(wx0[D_FEAT:].astype(bf16))
    w2x = (p['W_x'][1] / jnp.sqrt(jnp.float32(HID))).astype(bf16)
    w3x = (p['W_x'][2] / jnp.sqrt(jnp.float32(HID))).astype(bf16)

    x_out = pl.pallas_call(
        _edge_mlp_body,
        grid=(NB,),
        in_specs=[
            pl.BlockSpec((BLK, D_FEAT), lambda i: (i, 0)),
            pl.BlockSpec((BLK, 128), lambda i: (i, 0)),
            pl.BlockSpec((BLK, 1), lambda i: (i, 0)),
            pl.BlockSpec((D_FEAT, HID), lambda i: (0, 0)),
            pl.BlockSpec((128, HID), lambda i: (0, 0)),
            pl.BlockSpec((HID, HID), lambda i: (0, 0)),
            pl.BlockSpec((HID, HID), lambda i: (0, 0)),
        ],
        out_specs=pl.BlockSpec((BLK, HID), lambda i: (i, 0)),
        out_shape=jax.ShapeDtypeStruct((N_EDGES, HID), f32),
    )(x, w_edges, env_e, w1a, w1b, w2x, w3x)

    charges = q_pad[:N_NODES, 0]
    pot = pot_arr[0, 0]
    return x_out, V, charges, pot


# speedup vs baseline: 1.6565x; 1.0913x over previous
import functools

import jax
import jax.numpy as jnp
from jax import lax
from jax.experimental import pallas as pl
from jax.experimental.pallas import tpu as pltpu
from jax.experimental.pallas import tpu_sc as plsc

f32 = jnp.float32
bf16 = jnp.bfloat16

N_EDGES = 160000
N_NODES = 10000
NP = 10240
D_FEAT = 256
CE = 64
HID = 512

BLK = 1280
NB = N_EDGES // BLK

SC_ROWS = 1280
SC_COLS = 125
SC_TILES = 32
SC_RPT = SC_ROWS // SC_TILES

_EA = -28.0
_EB = 48.0
_EC = -21.0


def _chi_body(x_ref, w1_ref, w2_ref, w3_ref, chi_ref):
    h = jnp.dot(x_ref[:].astype(bf16), w1_ref[:], preferred_element_type=f32)
    h = h * jax.nn.sigmoid(h)
    h = jnp.dot(h.astype(bf16), w2_ref[:], preferred_element_type=f32)
    h = h * jax.nn.sigmoid(h)
    chi_ref[:] = jnp.sum(h * w3_ref[:], axis=1, keepdims=True)


def _node_body(p_ref, sp_ref, tab_ref, w0a_ref, w0b_ref, w1w_ref, sc_ref,
               q_ref, w_ref, pot_ref):
    chi_scale = sc_ref[0, 0]
    gamma_scale = sc_ref[0, 1]
    gamma_shift = sc_ref[0, 2]
    chis = (p_ref[:, 0:1] + p_ref[:, 1:2]) * chi_scale
    ks = lax.broadcasted_iota(jnp.int32, (NP, 128), 1)
    oh = (sp_ref[:] == ks).astype(f32)
    g = jnp.dot(oh, tab_ref[:], preferred_element_type=f32,
                precision=jax.lax.Precision.HIGHEST)
    ce = g[:, 0:CE]
    gam = g[:, CE:CE + 1] * gamma_scale + gamma_shift
    hraw = g[:, CE + 1:CE + 2]
    soft = jnp.maximum(hraw, 0.0) + jnp.log(1.0 + jnp.exp(-jnp.abs(hraw)))
    h_eff = soft + 1.0 / gam
    q = -chis / h_eff
    q_ref[:] = q
    pot_ref[:, :] = jnp.sum(0.5 * h_eff * q * q + chis * q).reshape(1, 1)
    h = q * w0a_ref[:] + jnp.dot(ce, w0b_ref[:], preferred_element_type=f32,
                                 precision=jax.lax.Precision.HIGHEST)
    h = h * jax.nn.sigmoid(h)
    w_ref[:] = jnp.dot(h, w1w_ref[:], preferred_element_type=f32,
                       precision=jax.lax.Precision.HIGHEST)


def _edge_mlp_body(x_ref, we_ref, v_ref, w1a_ref, w1b_ref, w2_ref, w3_ref,
                   out_ref):
    h = jnp.dot(x_ref[:].astype(bf16), w1a_ref[:], preferred_element_type=f32)
    h = h + jnp.dot(we_ref[:].astype(bf16), w1b_ref[:],
                    preferred_element_type=f32)
    h = h * jax.nn.sigmoid(h)
    h = jnp.dot(h.astype(bf16), w2_ref[:], preferred_element_type=f32)
    h = h * jax.nn.sigmoid(h)
    h = jnp.dot(h.astype(bf16), w3_ref[:], preferred_element_type=f32)
    v = v_ref[:]
    d = jnp.sqrt(jnp.sum(v * v, axis=1, keepdims=True))
    u = 1.0 + _EA * d**6 + _EB * d**7 + _EC * d**8
    env = jnp.where(d < 1.0, u, 0.0)
    out_ref[:] = env * h


SC_EPT = N_EDGES // SC_TILES
SC_GCHUNK = 128
SC_GFULL = SC_EPT // SC_GCHUNK
SC_GTAIL = SC_EPT - SC_GFULL * SC_GCHUNK


def _sc_scatter_add(chis2d, snd2d, zeros_np):
    mesh = plsc.VectorSubcoreMesh(core_axis_name="c", subcore_axis_name="s")

    @functools.partial(
        pl.kernel,
        out_type=jax.ShapeDtypeStruct((2 * NP,), f32),
        mesh=mesh,
        scratch_types=[
            pltpu.VMEM((SC_RPT, SC_COLS), jnp.int32),
            pltpu.VMEM((SC_RPT, SC_COLS), f32),
            pltpu.VMEM_SHARED((NP,), f32),
        ],
    )
    def _scatter_k(chis_hbm, snd_hbm, zero_hbm, out_hbm, idx_v, val_v, acc_sh):
        c = lax.axis_index("c")
        s = lax.axis_index("s")
        wid = c * 16 + s

        @pl.when(s == 0)
        def _():
            pltpu.sync_copy(zero_hbm, acc_sh)

        plsc.subcore_barrier()
        pltpu.sync_copy(snd_hbm.at[pl.ds(wid * SC_RPT, SC_RPT)], idx_v)
        pltpu.sync_copy(chis_hbm.at[pl.ds(wid * SC_RPT, SC_RPT)], val_v)

        def body(j, carry):
            pltpu.sync_copy(val_v.at[j], acc_sh.at[idx_v.at[j]], add=True)
            return carry

        lax.fori_loop(0, SC_RPT, body, 0)
        plsc.subcore_barrier()

        @pl.when(s == 0)
        def _():
            pltpu.sync_copy(acc_sh, out_hbm.at[pl.ds(c * NP, NP)])

    return _scatter_k(chis2d, snd2d, zeros_np)


def _sc_gather_rows(w_nodes, snd_flat):
    mesh = plsc.VectorSubcoreMesh(core_axis_name="c", subcore_axis_name="s")

    @functools.partial(
        pl.kernel,
        out_type=jax.ShapeDtypeStruct((N_EDGES, 128), f32),
        mesh=mesh,
        scratch_types=[
            pltpu.VMEM((SC_EPT,), jnp.int32),
            pltpu.VMEM((SC_GCHUNK, 128), f32),
            pltpu.VMEM((SC_GTAIL, 128), f32),
            pltpu.SemaphoreType.DMA,
        ],
    )
    def _gather_k(w_hbm, snd_hbm, out_hbm, idx_v, rows_v, tail_v, sem):
        c = lax.axis_index("c")
        s = lax.axis_index("s")
        wid = c * 16 + s
        base = wid * SC_EPT
        pltpu.sync_copy(snd_hbm.at[pl.ds(base, SC_EPT)], idx_v)

        def body(j, carry):
            pltpu.async_copy(
                w_hbm.at[idx_v.at[pl.ds(j * SC_GCHUNK, SC_GCHUNK)]],
                rows_v, sem).wait()
            pltpu.sync_copy(
                rows_v, out_hbm.at[pl.ds(base + j * SC_GCHUNK, SC_GCHUNK)])
            return carry

        lax.fori_loop(0, SC_GFULL, body, 0)
        pltpu.async_copy(
            w_hbm.at[idx_v.at[pl.ds(SC_GFULL * SC_GCHUNK, SC_GTAIL)]],
            tail_v, sem).wait()
        pltpu.sync_copy(
            tail_v,
            out_hbm.at[pl.ds(base + SC_GFULL * SC_GCHUNK, SC_GTAIL)])

    return _gather_k(w_nodes, snd_flat)


def kernel(vectors, x, V, params, senders, species):
    p = params

    w1c = (p['W_chi'][0] / jnp.sqrt(jnp.float32(D_FEAT))).astype(bf16)
    w2c = (p['W_chi'][1] / jnp.sqrt(jnp.float32(CE))).astype(bf16)
    w3c = (p['W_chi'][2][:, 0] / jnp.sqrt(jnp.float32(CE))).reshape(1, CE)

    chi_e = pl.pallas_call(
        _chi_body,
        grid=(NB,),
        in_specs=[
            pl.BlockSpec((BLK, D_FEAT), lambda i: (i, 0)),
            pl.BlockSpec((D_FEAT, CE), lambda i: (0, 0)),
            pl.BlockSpec((CE, CE), lambda i: (0, 0)),
            pl.BlockSpec((1, CE), lambda i: (0, 0)),
        ],
        out_specs=pl.BlockSpec((BLK, 1), lambda i: (i, 0)),
        out_shape=jax.ShapeDtypeStruct((N_EDGES, 1), f32),
    )(x, w1c, w2c, w3c)

    chis2d = chi_e.reshape(SC_ROWS, SC_COLS)
    snd2d = senders.reshape(SC_ROWS, SC_COLS)
    partials = _sc_scatter_add(chis2d, snd2d,
                               jnp.zeros((NP,), f32)).reshape(2, NP)

    table = jnp.concatenate([
        p['charge_embed'], p['radius'][:, None], p['hardness'][:, None],
        jnp.zeros((100, 128 - CE - 2), f32)], axis=1)
    table = jnp.concatenate([table, jnp.zeros((28, 128), f32)], axis=0)
    w0 = p['W_w'][0] / jnp.sqrt(jnp.float32(1 + CE))
    w0a = w0[0:1, :]
    w0b = w0[1:, :]
    w1w = jnp.concatenate([
        p['W_w'][1] / jnp.sqrt(jnp.float32(CE)),
        jnp.zeros((CE, 128 - CE), f32)], axis=1)
    scal = jnp.stack([p['chi_scale'], p['gamma_scale'],
                      p['gamma_shift']]).reshape(1, 3)
    sp_col = jnp.pad(species, (0, NP - N_NODES)).reshape(NP, 1)

    q_pad, w_nodes, pot_arr = pl.pallas_call(
        _node_body,
        grid=(1,),
        in_specs=[
            pl.BlockSpec((NP, 2), lambda i: (0, 0)),
            pl.BlockSpec((NP, 1), lambda i: (0, 0)),
            pl.BlockSpec((128, 128), lambda i: (0, 0)),
            pl.BlockSpec((1, CE), lambda i: (0, 0)),
            pl.BlockSpec((CE, CE), lambda i: (0, 0)),
            pl.BlockSpec((CE, 128), lambda i: (0, 0)),
            pl.BlockSpec((1, 3), lambda i: (0, 0)),
        ],
        out_specs=[
            pl.BlockSpec((NP, 1), lambda i: (0, 0)),
            pl.BlockSpec((NP, 128), lambda i: (0, 0)),
            pl.BlockSpec((1, 1), lambda i: (0, 0)),
        ],
        out_shape=[
            jax.ShapeDtypeStruct((NP, 1), f32),
            jax.ShapeDtypeStruct((NP, 128), f32),
            jax.ShapeDtypeStruct((1, 1), f32),
        ],
    )(partials.T, sp_col, table, w0a, w0b, w1w, scal)

    w_edges = _sc_gather_rows(w_nodes, senders)

    wx0 = p['W_x'][0] / jnp.sqrt(jnp.float32(D_FEAT + CE))
    w1a = wx0[:D_FEAT].astype(bf16)
    w1b = jnp.concatenate([wx0[D_FEAT:].astype(bf16),
                           jnp.zeros((128 - CE, HID), bf16)], axis=0)
    w2x = (p['W_x'][1] / jnp.sqrt(jnp.float32(HID))).astype(bf16)
    w3x = (p['W_x'][2] / jnp.sqrt(jnp.float32(HID))).astype(bf16)

    x_out = pl.pallas_call(
        _edge_mlp_body,
        grid=(NB,),
        in_specs=[
            pl.BlockSpec((BLK, D_FEAT), lambda i: (i, 0)),
            pl.BlockSpec((BLK, 128), lambda i: (i, 0)),
            pl.BlockSpec((BLK, 3), lambda i: (i, 0)),
            pl.BlockSpec((D_FEAT, HID), lambda i: (0, 0)),
            pl.BlockSpec((128, HID), lambda i: (0, 0)),
            pl.BlockSpec((HID, HID), lambda i: (0, 0)),
            pl.BlockSpec((HID, HID), lambda i: (0, 0)),
        ],
        out_specs=pl.BlockSpec((BLK, HID), lambda i: (i, 0)),
        out_shape=jax.ShapeDtypeStruct((N_EDGES, HID), f32),
    )(x, w_edges, vectors, w1a, w1b, w2x, w3x)

    charges = q_pad[:N_NODES, 0]
    pot = pot_arr[0, 0]
    return x_out, V, charges, pot


# speedup vs baseline: 1.7351x; 1.0474x over previous
import functools

import jax
import jax.numpy as jnp
from jax import lax
from jax.experimental import pallas as pl
from jax.experimental.pallas import tpu as pltpu
from jax.experimental.pallas import tpu_sc as plsc

f32 = jnp.float32
bf16 = jnp.bfloat16

N_EDGES = 160000
N_NODES = 10000
NP = 10240
D_FEAT = 256
CE = 64
HID = 512

BLK = 1280
NB = N_EDGES // BLK

SC_ROWS = 1280
SC_COLS = 125
SC_TILES = 32
SC_RPT = SC_ROWS // SC_TILES

_EA = -28.0
_EB = 48.0
_EC = -21.0


def _chi_body(x_ref, w1_ref, w2_ref, w3_ref, chi_ref):
    h = jnp.dot(x_ref[:].astype(bf16), w1_ref[:], preferred_element_type=f32)
    h = h * jax.nn.sigmoid(h)
    h = jnp.dot(h.astype(bf16), w2_ref[:], preferred_element_type=f32)
    h = h * jax.nn.sigmoid(h)
    chi_t = lax.dot_general(w3_ref[:], h.astype(bf16),
                            (((1,), (1,)), ((), ())),
                            preferred_element_type=f32)
    chi_ref[:] = chi_t.reshape(1, 1, BLK)


def _node_body(p_ref, sp_ref, tab_ref, w0a_ref, w0b_ref, w1w_ref, sc_ref,
               q_ref, w_ref, pot_ref):
    chi_scale = sc_ref[0, 0]
    gamma_scale = sc_ref[0, 1]
    gamma_shift = sc_ref[0, 2]
    chis = (p_ref[:, 0:1] + p_ref[:, 1:2]) * chi_scale
    ks = lax.broadcasted_iota(jnp.int32, (NP, 128), 1)
    oh = (sp_ref[:] == ks).astype(f32)
    g = jnp.dot(oh, tab_ref[:], preferred_element_type=f32,
                precision=jax.lax.Precision.HIGHEST)
    ce = g[:, 0:CE]
    gam = g[:, CE:CE + 1] * gamma_scale + gamma_shift
    hraw = g[:, CE + 1:CE + 2]
    soft = jnp.maximum(hraw, 0.0) + jnp.log(1.0 + jnp.exp(-jnp.abs(hraw)))
    h_eff = soft + 1.0 / gam
    q = -chis / h_eff
    q_ref[:] = q
    pot_ref[:, :] = jnp.sum(0.5 * h_eff * q * q + chis * q).reshape(1, 1)
    h = q * w0a_ref[:] + jnp.dot(ce, w0b_ref[:], preferred_element_type=f32,
                                 precision=jax.lax.Precision.HIGHEST)
    h = h * jax.nn.sigmoid(h)
    w_ref[:] = jnp.dot(h, w1w_ref[:], preferred_element_type=f32,
                       precision=jax.lax.Precision.HIGHEST)


def _edge_mlp_body(x_ref, we_ref, v_ref, w1a_ref, w1b_ref, w2_ref, w3_ref,
                   out_ref):
    h = jnp.dot(x_ref[:].astype(bf16), w1a_ref[:], preferred_element_type=f32)
    h = h + jnp.dot(we_ref[:].astype(bf16), w1b_ref[:],
                    preferred_element_type=f32)
    h = h * jax.nn.sigmoid(h)
    h = jnp.dot(h.astype(bf16), w2_ref[:], preferred_element_type=f32)
    h = h * jax.nn.sigmoid(h)
    h = jnp.dot(h.astype(bf16), w3_ref[:], preferred_element_type=f32)
    v = v_ref[:]
    d2 = lax.transpose(jnp.sum(v * v, axis=0, keepdims=True), (1, 0))
    d = jnp.sqrt(d2)
    u = 1.0 + _EA * d**6 + _EB * d**7 + _EC * d**8
    env = jnp.where(d < 1.0, u, 0.0)
    out_ref[:] = env * h


SC_EPT = N_EDGES // SC_TILES
SC_GCHUNK = 128
SC_GFULL = SC_EPT // SC_GCHUNK
SC_GTAIL = SC_EPT - SC_GFULL * SC_GCHUNK


def _sc_scatter_add(chis2d, snd2d, zeros_np):
    mesh = plsc.VectorSubcoreMesh(core_axis_name="c", subcore_axis_name="s")

    @functools.partial(
        pl.kernel,
        out_type=jax.ShapeDtypeStruct((2 * NP,), f32),
        mesh=mesh,
        scratch_types=[
            pltpu.VMEM((SC_RPT, SC_COLS), jnp.int32),
            pltpu.VMEM((SC_RPT, SC_COLS), f32),
            pltpu.VMEM_SHARED((NP,), f32),
        ],
    )
    def _scatter_k(chis_hbm, snd_hbm, zero_hbm, out_hbm, idx_v, val_v, acc_sh):
        c = lax.axis_index("c")
        s = lax.axis_index("s")
        wid = c * 16 + s

        @pl.when(s == 0)
        def _():
            pltpu.sync_copy(zero_hbm, acc_sh)

        plsc.subcore_barrier()
        pltpu.sync_copy(snd_hbm.at[pl.ds(wid * SC_RPT, SC_RPT)], idx_v)
        pltpu.sync_copy(chis_hbm.at[pl.ds(wid * SC_RPT, SC_RPT)], val_v)

        def body(j, carry):
            pltpu.sync_copy(val_v.at[j], acc_sh.at[idx_v.at[j]], add=True)
            return carry

        lax.fori_loop(0, SC_RPT, body, 0)
        plsc.subcore_barrier()

        @pl.when(s == 0)
        def _():
            pltpu.sync_copy(acc_sh, out_hbm.at[pl.ds(c * NP, NP)])

    return _scatter_k(chis2d, snd2d, zeros_np)


def _sc_gather_rows(w_nodes, snd_flat):
    mesh = plsc.VectorSubcoreMesh(core_axis_name="c", subcore_axis_name="s")

    @functools.partial(
        pl.kernel,
        out_type=jax.ShapeDtypeStruct((N_EDGES, 128), f32),
        mesh=mesh,
        scratch_types=[
            pltpu.VMEM((SC_EPT,), jnp.int32),
            pltpu.VMEM((SC_GCHUNK, 128), f32),
            pltpu.VMEM((SC_GTAIL, 128), f32),
            pltpu.SemaphoreType.DMA,
        ],
    )
    def _gather_k(w_hbm, snd_hbm, out_hbm, idx_v, rows_v, tail_v, sem):
        c = lax.axis_index("c")
        s = lax.axis_index("s")
        wid = c * 16 + s
        base = wid * SC_EPT
        pltpu.sync_copy(snd_hbm.at[pl.ds(base, SC_EPT)], idx_v)

        def body(j, carry):
            pltpu.async_copy(
                w_hbm.at[idx_v.at[pl.ds(j * SC_GCHUNK, SC_GCHUNK)]],
                rows_v, sem).wait()
            pltpu.sync_copy(
                rows_v, out_hbm.at[pl.ds(base + j * SC_GCHUNK, SC_GCHUNK)])
            return carry

        lax.fori_loop(0, SC_GFULL, body, 0)
        pltpu.async_copy(
            w_hbm.at[idx_v.at[pl.ds(SC_GFULL * SC_GCHUNK, SC_GTAIL)]],
            tail_v, sem).wait()
        pltpu.sync_copy(
            tail_v,
            out_hbm.at[pl.ds(base + SC_GFULL * SC_GCHUNK, SC_GTAIL)])

    return _gather_k(w_nodes, snd_flat)


def kernel(vectors, x, V, params, senders, species):
    p = params

    w1c = (p['W_chi'][0] / jnp.sqrt(jnp.float32(D_FEAT))).astype(bf16)
    w2c = (p['W_chi'][1] / jnp.sqrt(jnp.float32(CE))).astype(bf16)
    w3c = (p['W_chi'][2][:, 0] / jnp.sqrt(jnp.float32(CE))
           ).reshape(1, CE).astype(bf16)

    chi_e = pl.pallas_call(
        _chi_body,
        grid=(NB,),
        in_specs=[
            pl.BlockSpec((BLK, D_FEAT), lambda i: (i, 0)),
            pl.BlockSpec((D_FEAT, CE), lambda i: (0, 0)),
            pl.BlockSpec((CE, CE), lambda i: (0, 0)),
            pl.BlockSpec((1, CE), lambda i: (0, 0)),
        ],
        out_specs=pl.BlockSpec((1, 1, BLK), lambda i: (i, 0, 0)),
        out_shape=jax.ShapeDtypeStruct((NB, 1, BLK), f32),
    )(x, w1c, w2c, w3c)

    chis2d = chi_e.reshape(SC_ROWS, SC_COLS)
    snd2d = senders.reshape(SC_ROWS, SC_COLS)
    partials = _sc_scatter_add(chis2d, snd2d,
                               jnp.zeros((NP,), f32)).reshape(2, NP)

    table = jnp.concatenate([
        p['charge_embed'], p['radius'][:, None], p['hardness'][:, None],
        jnp.zeros((100, 128 - CE - 2), f32)], axis=1)
    table = jnp.concatenate([table, jnp.zeros((28, 128), f32)], axis=0)
    w0 = p['W_w'][0] / jnp.sqrt(jnp.float32(1 + CE))
    w0a = w0[0:1, :]
    w0b = w0[1:, :]
    w1w = jnp.concatenate([
        p['W_w'][1] / jnp.sqrt(jnp.float32(CE)),
        jnp.zeros((CE, 128 - CE), f32)], axis=1)
    scal = jnp.stack([p['chi_scale'], p['gamma_scale'],
                      p['gamma_shift']]).reshape(1, 3)
    sp_col = jnp.pad(species, (0, NP - N_NODES)).reshape(NP, 1)

    q_pad, w_nodes, pot_arr = pl.pallas_call(
        _node_body,
        grid=(1,),
        in_specs=[
            pl.BlockSpec((NP, 2), lambda i: (0, 0)),
            pl.BlockSpec((NP, 1), lambda i: (0, 0)),
            pl.BlockSpec((128, 128), lambda i: (0, 0)),
            pl.BlockSpec((1, CE), lambda i: (0, 0)),
            pl.BlockSpec((CE, CE), lambda i: (0, 0)),
            pl.BlockSpec((CE, 128), lambda i: (0, 0)),
            pl.BlockSpec((1, 3), lambda i: (0, 0)),
        ],
        out_specs=[
            pl.BlockSpec((NP, 1), lambda i: (0, 0)),
            pl.BlockSpec((NP, 128), lambda i: (0, 0)),
            pl.BlockSpec((1, 1), lambda i: (0, 0)),
        ],
        out_shape=[
            jax.ShapeDtypeStruct((NP, 1), f32),
            jax.ShapeDtypeStruct((NP, 128), f32),
            jax.ShapeDtypeStruct((1, 1), f32),
        ],
    )(partials.T, sp_col, table, w0a, w0b, w1w, scal)

    w_edges = _sc_gather_rows(w_nodes, senders)

    wx0 = p['W_x'][0] / jnp.sqrt(jnp.float32(D_FEAT + CE))
    w1a = wx0[:D_FEAT].astype(bf16)
    w1b = jnp.concatenate([wx0[D_FEAT:].astype(bf16),
                           jnp.zeros((128 - CE, HID), bf16)], axis=0)
    w2x = (p['W_x'][1] / jnp.sqrt(jnp.float32(HID))).astype(bf16)
    w3x = (p['W_x'][2] / jnp.sqrt(jnp.float32(HID))).astype(bf16)

    x_out = pl.pallas_call(
        _edge_mlp_body,
        grid=(NB,),
        in_specs=[
            pl.BlockSpec((BLK, D_FEAT), lambda i: (i, 0)),
            pl.BlockSpec((BLK, 128), lambda i: (i, 0)),
            pl.BlockSpec((3, BLK), lambda i: (0, i)),
            pl.BlockSpec((D_FEAT, HID), lambda i: (0, 0)),
            pl.BlockSpec((128, HID), lambda i: (0, 0)),
            pl.BlockSpec((HID, HID), lambda i: (0, 0)),
            pl.BlockSpec((HID, HID), lambda i: (0, 0)),
        ],
        out_specs=pl.BlockSpec((BLK, HID), lambda i: (i, 0)),
        out_shape=jax.ShapeDtypeStruct((N_EDGES, HID), f32),
    )(x, w_edges, vectors.T, w1a, w1b, w2x, w3x)

    charges = q_pad[:N_NODES, 0]
    pot = pot_arr[0, 0]
    return x_out, V, charges, pot


# speedup vs baseline: 1.8241x; 1.0513x over previous
import functools

import jax
import jax.numpy as jnp
from jax import lax
from jax.experimental import pallas as pl
from jax.experimental.pallas import tpu as pltpu
from jax.experimental.pallas import tpu_sc as plsc

f32 = jnp.float32
bf16 = jnp.bfloat16

N_EDGES = 160000
N_NODES = 10000
NP = 10240
D_FEAT = 256
CE = 64
HID = 512

BLK = 3200
NB = N_EDGES // BLK

SC_ROWS = 1280
SC_COLS = 125
SC_TILES = 32
SC_RPT = SC_ROWS // SC_TILES

_EA = -28.0
_EB = 48.0
_EC = -21.0


def _chi_body(x_ref, w1_ref, w2_ref, w3_ref, chi_ref):
    h = jnp.dot(x_ref[:].astype(bf16), w1_ref[:], preferred_element_type=f32)
    h = h * jax.nn.sigmoid(h)
    h = jnp.dot(h.astype(bf16), w2_ref[:], preferred_element_type=f32)
    h = h * jax.nn.sigmoid(h)
    chi_t = lax.dot_general(w3_ref[:], h.astype(bf16),
                            (((1,), (1,)), ((), ())),
                            preferred_element_type=f32)
    chi_ref[:] = chi_t.reshape(1, 1, BLK)


def _node_body(p_ref, sp_ref, tab_ref, w0a_ref, w0b_ref, w1w_ref, sc_ref,
               q_ref, w_ref, pot_ref):
    chi_scale = sc_ref[0, 0]
    gamma_scale = sc_ref[0, 1]
    gamma_shift = sc_ref[0, 2]
    chis = (p_ref[:, 0:1] + p_ref[:, 1:2]) * chi_scale
    ks = lax.broadcasted_iota(jnp.int32, (NP, 128), 1)
    oh = (sp_ref[:] == ks).astype(f32)
    g = jnp.dot(oh, tab_ref[:], preferred_element_type=f32,
                precision=jax.lax.Precision.HIGHEST)
    ce = g[:, 0:CE]
    gam = g[:, CE:CE + 1] * gamma_scale + gamma_shift
    hraw = g[:, CE + 1:CE + 2]
    soft = jnp.maximum(hraw, 0.0) + jnp.log(1.0 + jnp.exp(-jnp.abs(hraw)))
    h_eff = soft + 1.0 / gam
    q = -chis / h_eff
    q_ref[:] = q
    pot_ref[:, :] = jnp.sum(0.5 * h_eff * q * q + chis * q).reshape(1, 1)
    h = q * w0a_ref[:] + jnp.dot(ce, w0b_ref[:], preferred_element_type=f32,
                                 precision=jax.lax.Precision.HIGHEST)
    h = h * jax.nn.sigmoid(h)
    w_ref[:] = jnp.dot(h, w1w_ref[:], preferred_element_type=f32,
                       precision=jax.lax.Precision.HIGHEST)


def _edge_mlp_body(x_ref, we_ref, v_ref, w1a_ref, w1b_ref, w2_ref, w3_ref,
                   out_ref):
    HB = BLK // 2
    v = v_ref[:]
    d2 = lax.transpose(jnp.sum(v * v, axis=0, keepdims=True), (1, 0))
    d = jnp.sqrt(d2)
    u = 1.0 + _EA * d**6 + _EB * d**7 + _EC * d**8
    env = jnp.where(d < 1.0, u, 0.0)
    for k in range(2):
        a, b = k * HB, (k + 1) * HB
        h = jnp.dot(x_ref[a:b, :].astype(bf16), w1a_ref[:],
                    preferred_element_type=f32)
        h = h + jnp.dot(we_ref[a:b, :].astype(bf16), w1b_ref[:],
                        preferred_element_type=f32)
        h = h * jax.nn.sigmoid(h)
        h = jnp.dot(h.astype(bf16), w2_ref[:], preferred_element_type=f32)
        h = h * jax.nn.sigmoid(h)
        h = jnp.dot(h.astype(bf16), w3_ref[:], preferred_element_type=f32)
        out_ref[a:b, :] = env[a:b, :] * h


SC_EPT = N_EDGES // SC_TILES
SC_GCHUNK = 128
SC_GFULL = SC_EPT // SC_GCHUNK
SC_GTAIL = SC_EPT - SC_GFULL * SC_GCHUNK


def _sc_scatter_add(chis2d, snd2d, zeros_np):
    mesh = plsc.VectorSubcoreMesh(core_axis_name="c", subcore_axis_name="s")

    @functools.partial(
        pl.kernel,
        out_type=jax.ShapeDtypeStruct((2 * NP,), f32),
        mesh=mesh,
        scratch_types=[
            pltpu.VMEM((SC_RPT, SC_COLS), jnp.int32),
            pltpu.VMEM((SC_RPT, SC_COLS), f32),
            pltpu.VMEM_SHARED((NP,), f32),
        ],
    )
    def _scatter_k(chis_hbm, snd_hbm, zero_hbm, out_hbm, idx_v, val_v, acc_sh):
        c = lax.axis_index("c")
        s = lax.axis_index("s")
        wid = c * 16 + s

        @pl.when(s == 0)
        def _():
            pltpu.sync_copy(zero_hbm, acc_sh)

        plsc.subcore_barrier()
        pltpu.sync_copy(snd_hbm.at[pl.ds(wid * SC_RPT, SC_RPT)], idx_v)
        pltpu.sync_copy(chis_hbm.at[pl.ds(wid * SC_RPT, SC_RPT)], val_v)

        def body(j, carry):
            pltpu.sync_copy(val_v.at[j], acc_sh.at[idx_v.at[j]], add=True)
            return carry

        lax.fori_loop(0, SC_RPT, body, 0)
        plsc.subcore_barrier()

        @pl.when(s == 0)
        def _():
            pltpu.sync_copy(acc_sh, out_hbm.at[pl.ds(c * NP, NP)])

    return _scatter_k(chis2d, snd2d, zeros_np)


def _sc_gather_rows(w_nodes, snd_flat):
    mesh = plsc.VectorSubcoreMesh(core_axis_name="c", subcore_axis_name="s")

    @functools.partial(
        pl.kernel,
        out_type=jax.ShapeDtypeStruct((N_EDGES, 128), f32),
        mesh=mesh,
        scratch_types=[
            pltpu.VMEM((SC_EPT,), jnp.int32),
            pltpu.VMEM((SC_GCHUNK, 128), f32),
            pltpu.VMEM((SC_GTAIL, 128), f32),
            pltpu.SemaphoreType.DMA,
        ],
    )
    def _gather_k(w_hbm, snd_hbm, out_hbm, idx_v, rows_v, tail_v, sem):
        c = lax.axis_index("c")
        s = lax.axis_index("s")
        wid = c * 16 + s
        base = wid * SC_EPT
        pltpu.sync_copy(snd_hbm.at[pl.ds(base, SC_EPT)], idx_v)

        def body(j, carry):
            pltpu.async_copy(
                w_hbm.at[idx_v.at[pl.ds(j * SC_GCHUNK, SC_GCHUNK)]],
                rows_v, sem).wait()
            pltpu.sync_copy(
                rows_v, out_hbm.at[pl.ds(base + j * SC_GCHUNK, SC_GCHUNK)])
            return carry

        lax.fori_loop(0, SC_GFULL, body, 0)
        pltpu.async_copy(
            w_hbm.at[idx_v.at[pl.ds(SC_GFULL * SC_GCHUNK, SC_GTAIL)]],
            tail_v, sem).wait()
        pltpu.sync_copy(
            tail_v,
            out_hbm.at[pl.ds(base + SC_GFULL * SC_GCHUNK, SC_GTAIL)])

    return _gather_k(w_nodes, snd_flat)


def kernel(vectors, x, V, params, senders, species):
    p = params

    w1c = (p['W_chi'][0] / jnp.sqrt(jnp.float32(D_FEAT))).astype(bf16)
    w2c = (p['W_chi'][1] / jnp.sqrt(jnp.float32(CE))).astype(bf16)
    w3c = (p['W_chi'][2][:, 0] / jnp.sqrt(jnp.float32(CE))
           ).reshape(1, CE).astype(bf16)

    chi_e = pl.pallas_call(
        _chi_body,
        grid=(NB,),
        in_specs=[
            pl.BlockSpec((BLK, D_FEAT), lambda i: (i, 0)),
            pl.BlockSpec((D_FEAT, CE), lambda i: (0, 0)),
            pl.BlockSpec((CE, CE), lambda i: (0, 0)),
            pl.BlockSpec((1, CE), lambda i: (0, 0)),
        ],
        out_specs=pl.BlockSpec((1, 1, BLK), lambda i: (i, 0, 0)),
        out_shape=jax.ShapeDtypeStruct((NB, 1, BLK), f32),
    )(x, w1c, w2c, w3c)

    chis2d = chi_e.reshape(SC_ROWS, SC_COLS)
    snd2d = senders.reshape(SC_ROWS, SC_COLS)
    partials = _sc_scatter_add(chis2d, snd2d,
                               jnp.zeros((NP,), f32)).reshape(2, NP)

    table = jnp.concatenate([
        p['charge_embed'], p['radius'][:, None], p['hardness'][:, None],
        jnp.zeros((100, 128 - CE - 2), f32)], axis=1)
    table = jnp.concatenate([table, jnp.zeros((28, 128), f32)], axis=0)
    w0 = p['W_w'][0] / jnp.sqrt(jnp.float32(1 + CE))
    w0a = w0[0:1, :]
    w0b = w0[1:, :]
    w1w = jnp.concatenate([
        p['W_w'][1] / jnp.sqrt(jnp.float32(CE)),
        jnp.zeros((CE, 128 - CE), f32)], axis=1)
    scal = jnp.stack([p['chi_scale'], p['gamma_scale'],
                      p['gamma_shift']]).reshape(1, 3)
    sp_col = jnp.pad(species, (0, NP - N_NODES)).reshape(NP, 1)

    q_pad, w_nodes, pot_arr = pl.pallas_call(
        _node_body,
        grid=(1,),
        in_specs=[
            pl.BlockSpec((NP, 2), lambda i: (0, 0)),
            pl.BlockSpec((NP, 1), lambda i: (0, 0)),
            pl.BlockSpec((128, 128), lambda i: (0, 0)),
            pl.BlockSpec((1, CE), lambda i: (0, 0)),
            pl.BlockSpec((CE, CE), lambda i: (0, 0)),
            pl.BlockSpec((CE, 128), lambda i: (0, 0)),
            pl.BlockSpec((1, 3), lambda i: (0, 0)),
        ],
        out_specs=[
            pl.BlockSpec((NP, 1), lambda i: (0, 0)),
            pl.BlockSpec((NP, 128), lambda i: (0, 0)),
            pl.BlockSpec((1, 1), lambda i: (0, 0)),
        ],
        out_shape=[
            jax.ShapeDtypeStruct((NP, 1), f32),
            jax.ShapeDtypeStruct((NP, 128), f32),
            jax.ShapeDtypeStruct((1, 1), f32),
        ],
    )(partials.T, sp_col, table, w0a, w0b, w1w, scal)

    w_edges = _sc_gather_rows(w_nodes, senders)

    wx0 = p['W_x'][0] / jnp.sqrt(jnp.float32(D_FEAT + CE))
    w1a = wx0[:D_FEAT].astype(bf16)
    w1b = jnp.concatenate([wx0[D_FEAT:].astype(bf16),
                           jnp.zeros((128 - CE, HID), bf16)], axis=0)
    w2x = (p['W_x'][1] / jnp.sqrt(jnp.float32(HID))).astype(bf16)
    w3x = (p['W_x'][2] / jnp.sqrt(jnp.float32(HID))).astype(bf16)

    x_out = pl.pallas_call(
        _edge_mlp_body,
        grid=(NB,),
        in_specs=[
            pl.BlockSpec((BLK, D_FEAT), lambda i: (i, 0)),
            pl.BlockSpec((BLK, 128), lambda i: (i, 0)),
            pl.BlockSpec((3, BLK), lambda i: (0, i)),
            pl.BlockSpec((D_FEAT, HID), lambda i: (0, 0)),
            pl.BlockSpec((128, HID), lambda i: (0, 0)),
            pl.BlockSpec((HID, HID), lambda i: (0, 0)),
            pl.BlockSpec((HID, HID), lambda i: (0, 0)),
        ],
        out_specs=pl.BlockSpec((BLK, HID), lambda i: (i, 0)),
        out_shape=jax.ShapeDtypeStruct((N_EDGES, HID), f32),
    )(x, w_edges, vectors.T, w1a, w1b, w2x, w3x)

    charges = q_pad[:N_NODES, 0]
    pot = pot_arr[0, 0]
    return x_out, V, charges, pot


# speedup vs baseline: 1.9103x; 1.0472x over previous
import functools

import jax
import jax.numpy as jnp
from jax import lax
from jax.experimental import pallas as pl
from jax.experimental.pallas import tpu as pltpu
from jax.experimental.pallas import tpu_sc as plsc

f32 = jnp.float32
bf16 = jnp.bfloat16

N_EDGES = 160000
N_NODES = 10000
NP = 10240
D_FEAT = 256
CE = 64
HID = 512

BLK = 3200
NB = N_EDGES // BLK

SC_ROWS = 1280
SC_COLS = 125
SC_TILES = 32
SC_RPT = SC_ROWS // SC_TILES

_EA = -28.0
_EB = 48.0
_EC = -21.0


def _chi_body(x_ref, w1_ref, w2_ref, w3_ref, chi_ref):
    h = jnp.dot(x_ref[:].astype(bf16), w1_ref[:], preferred_element_type=f32)
    h = h * jax.nn.sigmoid(h)
    h = jnp.dot(h.astype(bf16), w2_ref[:], preferred_element_type=f32)
    h = h * jax.nn.sigmoid(h)
    chi_t = lax.dot_general(w3_ref[:], h.astype(bf16),
                            (((1,), (1,)), ((), ())),
                            preferred_element_type=f32)
    chi_ref[:] = chi_t.reshape(1, 1, BLK)


def _node_body(p_ref, sp_ref, tab_ref, w0a_ref, w0b_ref, w1w_ref, sc_ref,
               q_ref, w_ref, pot_ref):
    chi_scale = sc_ref[0, 0]
    gamma_scale = sc_ref[0, 1]
    gamma_shift = sc_ref[0, 2]
    chis = (p_ref[:, 0:1] + p_ref[:, 1:2]) * chi_scale
    ks = lax.broadcasted_iota(jnp.int32, (NP, 128), 1)
    oh = (sp_ref[:] == ks).astype(f32)
    g = jnp.dot(oh, tab_ref[:], preferred_element_type=f32,
                precision=jax.lax.Precision.HIGHEST)
    ce = g[:, 0:CE]
    gam = g[:, CE:CE + 1] * gamma_scale + gamma_shift
    hraw = g[:, CE + 1:CE + 2]
    soft = jnp.maximum(hraw, 0.0) + jnp.log(1.0 + jnp.exp(-jnp.abs(hraw)))
    h_eff = soft + 1.0 / gam
    q = -chis / h_eff
    q_ref[:] = q
    pot_ref[:, :] = jnp.sum(0.5 * h_eff * q * q + chis * q).reshape(1, 1)
    h = q * w0a_ref[:] + jnp.dot(ce, w0b_ref[:], preferred_element_type=f32,
                                 precision=jax.lax.Precision.HIGHEST)
    h = h * jax.nn.sigmoid(h)
    w_ref[:] = jnp.dot(h, w1w_ref[:], preferred_element_type=f32,
                       precision=jax.lax.Precision.HIGHEST)


def _edge_mlp_body(x_ref, we_ref, v_ref, w1a_ref, w1b_ref, w2_ref, w3_ref,
                   out_ref):
    HB = BLK // 2
    v = v_ref[:]
    d2 = lax.transpose(jnp.sum(v * v, axis=0, keepdims=True), (1, 0))
    d = jnp.sqrt(d2)
    u = 1.0 + _EA * d**6 + _EB * d**7 + _EC * d**8
    env = jnp.where(d < 1.0, u, 0.0)
    for k in range(2):
        a, b = k * HB, (k + 1) * HB
        h = jnp.dot(x_ref[a:b, :].astype(bf16), w1a_ref[:],
                    preferred_element_type=f32)
        h = (h + jnp.dot(we_ref[a:b, :].astype(bf16), w1b_ref[:],
                         preferred_element_type=f32)).astype(bf16)
        h = h * jax.nn.sigmoid(h)
        h = jnp.dot(h, w2_ref[:], preferred_element_type=f32).astype(bf16)
        h = h * jax.nn.sigmoid(h)
        h = jnp.dot(h, w3_ref[:], preferred_element_type=f32)
        out_ref[a:b, :] = env[a:b, :] * h


SC_EPT = N_EDGES // SC_TILES
SC_GCHUNK = 128
SC_GFULL = SC_EPT // SC_GCHUNK
SC_GTAIL = SC_EPT - SC_GFULL * SC_GCHUNK


def _sc_scatter_add(chis2d, snd2d, zeros_np):
    mesh = plsc.VectorSubcoreMesh(core_axis_name="c", subcore_axis_name="s")

    @functools.partial(
        pl.kernel,
        out_type=jax.ShapeDtypeStruct((2 * NP,), f32),
        mesh=mesh,
        scratch_types=[
            pltpu.VMEM((SC_RPT, SC_COLS), jnp.int32),
            pltpu.VMEM((SC_RPT, SC_COLS), f32),
            pltpu.VMEM_SHARED((NP,), f32),
        ],
    )
    def _scatter_k(chis_hbm, snd_hbm, zero_hbm, out_hbm, idx_v, val_v, acc_sh):
        c = lax.axis_index("c")
        s = lax.axis_index("s")
        wid = c * 16 + s

        @pl.when(s == 0)
        def _():
            pltpu.sync_copy(zero_hbm, acc_sh)

        plsc.subcore_barrier()
        pltpu.sync_copy(snd_hbm.at[pl.ds(wid * SC_RPT, SC_RPT)], idx_v)
        pltpu.sync_copy(chis_hbm.at[pl.ds(wid * SC_RPT, SC_RPT)], val_v)

        def body(j, carry):
            pltpu.sync_copy(val_v.at[j], acc_sh.at[idx_v.at[j]], add=True)
            return carry

        lax.fori_loop(0, SC_RPT, body, 0)
        plsc.subcore_barrier()

        @pl.when(s == 0)
        def _():
            pltpu.sync_copy(acc_sh, out_hbm.at[pl.ds(c * NP, NP)])

    return _scatter_k(chis2d, snd2d, zeros_np)


def _sc_gather_rows(w_nodes, snd_flat):
    mesh = plsc.VectorSubcoreMesh(core_axis_name="c", subcore_axis_name="s")

    @functools.partial(
        pl.kernel,
        out_type=jax.ShapeDtypeStruct((N_EDGES, 128), f32),
        mesh=mesh,
        scratch_types=[
            pltpu.VMEM((SC_EPT,), jnp.int32),
            pltpu.VMEM((SC_GCHUNK, 128), f32),
            pltpu.VMEM((SC_GTAIL, 128), f32),
            pltpu.SemaphoreType.DMA,
        ],
    )
    def _gather_k(w_hbm, snd_hbm, out_hbm, idx_v, rows_v, tail_v, sem):
        c = lax.axis_index("c")
        s = lax.axis_index("s")
        wid = c * 16 + s
        base = wid * SC_EPT
        pltpu.sync_copy(snd_hbm.at[pl.ds(base, SC_EPT)], idx_v)

        def body(j, carry):
            pltpu.async_copy(
                w_hbm.at[idx_v.at[pl.ds(j * SC_GCHUNK, SC_GCHUNK)]],
                rows_v, sem).wait()
            pltpu.sync_copy(
                rows_v, out_hbm.at[pl.ds(base + j * SC_GCHUNK, SC_GCHUNK)])
            return carry

        lax.fori_loop(0, SC_GFULL, body, 0)
        pltpu.async_copy(
            w_hbm.at[idx_v.at[pl.ds(SC_GFULL * SC_GCHUNK, SC_GTAIL)]],
            tail_v, sem).wait()
        pltpu.sync_copy(
            tail_v,
            out_hbm.at[pl.ds(base + SC_GFULL * SC_GCHUNK, SC_GTAIL)])

    return _gather_k(w_nodes, snd_flat)


def kernel(vectors, x, V, params, senders, species):
    p = params

    w1c = (p['W_chi'][0] / jnp.sqrt(jnp.float32(D_FEAT))).astype(bf16)
    w2c = (p['W_chi'][1] / jnp.sqrt(jnp.float32(CE))).astype(bf16)
    w3c = (p['W_chi'][2][:, 0] / jnp.sqrt(jnp.float32(CE))
           ).reshape(1, CE).astype(bf16)

    chi_e = pl.pallas_call(
        _chi_body,
        grid=(NB,),
        in_specs=[
            pl.BlockSpec((BLK, D_FEAT), lambda i: (i, 0)),
            pl.BlockSpec((D_FEAT, CE), lambda i: (0, 0)),
            pl.BlockSpec((CE, CE), lambda i: (0, 0)),
            pl.BlockSpec((1, CE), lambda i: (0, 0)),
        ],
        out_specs=pl.BlockSpec((1, 1, BLK), lambda i: (i, 0, 0)),
        out_shape=jax.ShapeDtypeStruct((NB, 1, BLK), f32),
    )(x, w1c, w2c, w3c)

    chis2d = chi_e.reshape(SC_ROWS, SC_COLS)
    snd2d = senders.reshape(SC_ROWS, SC_COLS)
    partials = _sc_scatter_add(chis2d, snd2d,
                               jnp.zeros((NP,), f32)).reshape(2, NP)

    table = jnp.concatenate([
        p['charge_embed'], p['radius'][:, None], p['hardness'][:, None],
        jnp.zeros((100, 128 - CE - 2), f32)], axis=1)
    table = jnp.concatenate([table, jnp.zeros((28, 128), f32)], axis=0)
    w0 = p['W_w'][0] / jnp.sqrt(jnp.float32(1 + CE))
    w0a = w0[0:1, :]
    w0b = w0[1:, :]
    w1w = jnp.concatenate([
        p['W_w'][1] / jnp.sqrt(jnp.float32(CE)),
        jnp.zeros((CE, 128 - CE), f32)], axis=1)
    scal = jnp.stack([p['chi_scale'], p['gamma_scale'],
                      p['gamma_shift']]).reshape(1, 3)
    sp_col = jnp.pad(species, (0, NP - N_NODES)).reshape(NP, 1)

    q_pad, w_nodes, pot_arr = pl.pallas_call(
        _node_body,
        grid=(1,),
        in_specs=[
            pl.BlockSpec((NP, 2), lambda i: (0, 0)),
            pl.BlockSpec((NP, 1), lambda i: (0, 0)),
            pl.BlockSpec((128, 128), lambda i: (0, 0)),
            pl.BlockSpec((1, CE), lambda i: (0, 0)),
            pl.BlockSpec((CE, CE), lambda i: (0, 0)),
            pl.BlockSpec((CE, 128), lambda i: (0, 0)),
            pl.BlockSpec((1, 3), lambda i: (0, 0)),
        ],
        out_specs=[
            pl.BlockSpec((NP, 1), lambda i: (0, 0)),
            pl.BlockSpec((NP, 128), lambda i: (0, 0)),
            pl.BlockSpec((1, 1), lambda i: (0, 0)),
        ],
        out_shape=[
            jax.ShapeDtypeStruct((NP, 1), f32),
            jax.ShapeDtypeStruct((NP, 128), f32),
            jax.ShapeDtypeStruct((1, 1), f32),
        ],
    )(partials.T, sp_col, table, w0a, w0b, w1w, scal)

    w_edges = _sc_gather_rows(w_nodes, senders)

    wx0 = p['W_x'][0] / jnp.sqrt(jnp.float32(D_FEAT + CE))
    w1a = wx0[:D_FEAT].astype(bf16)
    w1b = jnp.concatenate([wx0[D_FEAT:].astype(bf16),
                           jnp.zeros((128 - CE, HID), bf16)], axis=0)
    w2x = (p['W_x'][1] / jnp.sqrt(jnp.float32(HID))).astype(bf16)
    w3x = (p['W_x'][2] / jnp.sqrt(jnp.float32(HID))).astype(bf16)

    x_out = pl.pallas_call(
        _edge_mlp_body,
        grid=(NB,),
        in_specs=[
            pl.BlockSpec((BLK, D_FEAT), lambda i: (i, 0)),
            pl.BlockSpec((BLK, 128), lambda i: (i, 0)),
            pl.BlockSpec((3, BLK), lambda i: (0, i)),
            pl.BlockSpec((D_FEAT, HID), lambda i: (0, 0)),
            pl.BlockSpec((128, HID), lambda i: (0, 0)),
            pl.BlockSpec((HID, HID), lambda i: (0, 0)),
            pl.BlockSpec((HID, HID), lambda i: (0, 0)),
        ],
        out_specs=pl.BlockSpec((BLK, HID), lambda i: (i, 0)),
        out_shape=jax.ShapeDtypeStruct((N_EDGES, HID), f32),
    )(x, w_edges, vectors.T, w1a, w1b, w2x, w3x)

    charges = q_pad[:N_NODES, 0]
    pot = pot_arr[0, 0]
    return x_out, V, charges, pot


# speedup vs baseline: 1.9305x; 1.0106x over previous
import functools

import jax
import jax.numpy as jnp
from jax import lax
from jax.experimental import pallas as pl
from jax.experimental.pallas import tpu as pltpu
from jax.experimental.pallas import tpu_sc as plsc

f32 = jnp.float32
bf16 = jnp.bfloat16

N_EDGES = 160000
N_NODES = 10000
NP = 10240
D_FEAT = 256
CE = 64
HID = 512

BLK = 3200
NB = N_EDGES // BLK

SC_ROWS = 1280
SC_COLS = 125
SC_TILES = 32
SC_RPT = SC_ROWS // SC_TILES

_EA = -28.0
_EB = 48.0
_EC = -21.0


def _chi_body(x_ref, w1_ref, w2_ref, w3_ref, chi_ref):
    h = jnp.dot(x_ref[:].astype(bf16), w1_ref[:], preferred_element_type=f32)
    h = h * jax.nn.sigmoid(h)
    h = jnp.dot(h.astype(bf16), w2_ref[:], preferred_element_type=f32)
    h = h * jax.nn.sigmoid(h)
    chi_t = lax.dot_general(w3_ref[:], h.astype(bf16),
                            (((1,), (1,)), ((), ())),
                            preferred_element_type=f32)
    chi_ref[:] = chi_t.reshape(1, 1, BLK)


def _node_body(p_ref, sp_ref, tab_ref, w0a_ref, w0b_ref, w1w_ref, sc_ref,
               q_ref, w_ref, pot_ref):
    chi_scale = sc_ref[0, 0]
    gamma_scale = sc_ref[0, 1]
    gamma_shift = sc_ref[0, 2]
    chis = (p_ref[:, 0:1] + p_ref[:, 1:2]) * chi_scale
    ks = lax.broadcasted_iota(jnp.int32, (NP, 128), 1)
    oh = (sp_ref[:] == ks).astype(f32)
    g = jnp.dot(oh, tab_ref[:], preferred_element_type=f32,
                precision=jax.lax.Precision.HIGHEST)
    ce = g[:, 0:CE]
    gam = g[:, CE:CE + 1] * gamma_scale + gamma_shift
    hraw = g[:, CE + 1:CE + 2]
    soft = jnp.maximum(hraw, 0.0) + jnp.log(1.0 + jnp.exp(-jnp.abs(hraw)))
    h_eff = soft + 1.0 / gam
    q = -chis / h_eff
    q_ref[:] = q
    pot_ref[:, :] = jnp.sum(0.5 * h_eff * q * q + chis * q).reshape(1, 1)
    h = q * w0a_ref[:] + jnp.dot(ce, w0b_ref[:], preferred_element_type=f32,
                                 precision=jax.lax.Precision.HIGHEST)
    h = h * jax.nn.sigmoid(h)
    w_ref[:] = jnp.dot(h, w1w_ref[:], preferred_element_type=f32,
                       precision=jax.lax.Precision.HIGHEST)


def _edge_mlp_body(x_ref, we_ref, v_ref, w1a_ref, w1b_ref, w2_ref, w3_ref,
                   out_ref):
    HB = BLK // 4
    v = v_ref[:]
    d2 = lax.transpose(jnp.sum(v * v, axis=0, keepdims=True), (1, 0))
    d = jnp.sqrt(d2)
    u = 1.0 + _EA * d**6 + _EB * d**7 + _EC * d**8
    env = jnp.where(d < 1.0, u, 0.0)
    for k in range(4):
        a, b = k * HB, (k + 1) * HB
        h = jnp.dot(x_ref[a:b, :].astype(bf16), w1a_ref[:],
                    preferred_element_type=f32)
        h = (h + jnp.dot(we_ref[a:b, :].astype(bf16), w1b_ref[:],
                         preferred_element_type=f32)).astype(bf16)
        h = h * jax.nn.sigmoid(h)
        h = jnp.dot(h, w2_ref[:], preferred_element_type=f32).astype(bf16)
        h = h * jax.nn.sigmoid(h)
        h = jnp.dot(h, w3_ref[:], preferred_element_type=f32)
        out_ref[a:b, :] = env[a:b, :] * h


SC_EPT = N_EDGES // SC_TILES
SC_GCHUNK = 128
SC_GFULL = SC_EPT // SC_GCHUNK
SC_GTAIL = SC_EPT - SC_GFULL * SC_GCHUNK


def _sc_scatter_add(chis2d, snd2d, zeros_np):
    mesh = plsc.VectorSubcoreMesh(core_axis_name="c", subcore_axis_name="s")

    @functools.partial(
        pl.kernel,
        out_type=jax.ShapeDtypeStruct((2 * NP,), f32),
        mesh=mesh,
        scratch_types=[
            pltpu.VMEM((SC_RPT, SC_COLS), jnp.int32),
            pltpu.VMEM((SC_RPT, SC_COLS), f32),
            pltpu.VMEM_SHARED((NP,), f32),
        ],
    )
    def _scatter_k(chis_hbm, snd_hbm, zero_hbm, out_hbm, idx_v, val_v, acc_sh):
        c = lax.axis_index("c")
        s = lax.axis_index("s")
        wid = c * 16 + s

        @pl.when(s == 0)
        def _():
            pltpu.sync_copy(zero_hbm, acc_sh)

        plsc.subcore_barrier()
        pltpu.sync_copy(snd_hbm.at[pl.ds(wid * SC_RPT, SC_RPT)], idx_v)
        pltpu.sync_copy(chis_hbm.at[pl.ds(wid * SC_RPT, SC_RPT)], val_v)

        def body(j, carry):
            pltpu.sync_copy(val_v.at[j], acc_sh.at[idx_v.at[j]], add=True)
            return carry

        lax.fori_loop(0, SC_RPT, body, 0)
        plsc.subcore_barrier()

        @pl.when(s == 0)
        def _():
            pltpu.sync_copy(acc_sh, out_hbm.at[pl.ds(c * NP, NP)])

    return _scatter_k(chis2d, snd2d, zeros_np)


def _sc_gather_rows(w_nodes, snd_flat):
    mesh = plsc.VectorSubcoreMesh(core_axis_name="c", subcore_axis_name="s")

    @functools.partial(
        pl.kernel,
        out_type=jax.ShapeDtypeStruct((N_EDGES, 128), f32),
        mesh=mesh,
        scratch_types=[
            pltpu.VMEM((SC_EPT,), jnp.int32),
            pltpu.VMEM((SC_GCHUNK, 128), f32),
            pltpu.VMEM((SC_GTAIL, 128), f32),
            pltpu.SemaphoreType.DMA,
        ],
    )
    def _gather_k(w_hbm, snd_hbm, out_hbm, idx_v, rows_v, tail_v, sem):
        c = lax.axis_index("c")
        s = lax.axis_index("s")
        wid = c * 16 + s
        base = wid * SC_EPT
        pltpu.sync_copy(snd_hbm.at[pl.ds(base, SC_EPT)], idx_v)

        def body(j, carry):
            pltpu.async_copy(
                w_hbm.at[idx_v.at[pl.ds(j * SC_GCHUNK, SC_GCHUNK)]],
                rows_v, sem).wait()
            pltpu.sync_copy(
                rows_v, out_hbm.at[pl.ds(base + j * SC_GCHUNK, SC_GCHUNK)])
            return carry

        lax.fori_loop(0, SC_GFULL, body, 0)
        pltpu.async_copy(
            w_hbm.at[idx_v.at[pl.ds(SC_GFULL * SC_GCHUNK, SC_GTAIL)]],
            tail_v, sem).wait()
        pltpu.sync_copy(
            tail_v,
            out_hbm.at[pl.ds(base + SC_GFULL * SC_GCHUNK, SC_GTAIL)])

    return _gather_k(w_nodes, snd_flat)


def kernel(vectors, x, V, params, senders, species):
    p = params

    w1c = (p['W_chi'][0] / jnp.sqrt(jnp.float32(D_FEAT))).astype(bf16)
    w2c = (p['W_chi'][1] / jnp.sqrt(jnp.float32(CE))).astype(bf16)
    w3c = (p['W_chi'][2][:, 0] / jnp.sqrt(jnp.float32(CE))
           ).reshape(1, CE).astype(bf16)

    chi_e = pl.pallas_call(
        _chi_body,
        grid=(NB,),
        in_specs=[
            pl.BlockSpec((BLK, D_FEAT), lambda i: (i, 0)),
            pl.BlockSpec((D_FEAT, CE), lambda i: (0, 0)),
            pl.BlockSpec((CE, CE), lambda i: (0, 0)),
            pl.BlockSpec((1, CE), lambda i: (0, 0)),
        ],
        out_specs=pl.BlockSpec((1, 1, BLK), lambda i: (i, 0, 0)),
        out_shape=jax.ShapeDtypeStruct((NB, 1, BLK), f32),
    )(x, w1c, w2c, w3c)

    chis2d = chi_e.reshape(SC_ROWS, SC_COLS)
    snd2d = senders.reshape(SC_ROWS, SC_COLS)
    partials = _sc_scatter_add(chis2d, snd2d,
                               jnp.zeros((NP,), f32)).reshape(2, NP)

    table = jnp.concatenate([
        p['charge_embed'], p['radius'][:, None], p['hardness'][:, None],
        jnp.zeros((100, 128 - CE - 2), f32)], axis=1)
    table = jnp.concatenate([table, jnp.zeros((28, 128), f32)], axis=0)
    w0 = p['W_w'][0] / jnp.sqrt(jnp.float32(1 + CE))
    w0a = w0[0:1, :]
    w0b = w0[1:, :]
    w1w = jnp.concatenate([
        p['W_w'][1] / jnp.sqrt(jnp.float32(CE)),
        jnp.zeros((CE, 128 - CE), f32)], axis=1)
    scal = jnp.stack([p['chi_scale'], p['gamma_scale'],
                      p['gamma_shift']]).reshape(1, 3)
    sp_col = jnp.pad(species, (0, NP - N_NODES)).reshape(NP, 1)

    q_pad, w_nodes, pot_arr = pl.pallas_call(
        _node_body,
        grid=(1,),
        in_specs=[
            pl.BlockSpec((NP, 2), lambda i: (0, 0)),
            pl.BlockSpec((NP, 1), lambda i: (0, 0)),
            pl.BlockSpec((128, 128), lambda i: (0, 0)),
            pl.BlockSpec((1, CE), lambda i: (0, 0)),
            pl.BlockSpec((CE, CE), lambda i: (0, 0)),
            pl.BlockSpec((CE, 128), lambda i: (0, 0)),
            pl.BlockSpec((1, 3), lambda i: (0, 0)),
        ],
        out_specs=[
            pl.BlockSpec((NP, 1), lambda i: (0, 0)),
            pl.BlockSpec((NP, 128), lambda i: (0, 0)),
            pl.BlockSpec((1, 1), lambda i: (0, 0)),
        ],
        out_shape=[
            jax.ShapeDtypeStruct((NP, 1), f32),
            jax.ShapeDtypeStruct((NP, 128), f32),
            jax.ShapeDtypeStruct((1, 1), f32),
        ],
    )(partials.T, sp_col, table, w0a, w0b, w1w, scal)

    w_edges = _sc_gather_rows(w_nodes, senders)

    wx0 = p['W_x'][0] / jnp.sqrt(jnp.float32(D_FEAT + CE))
    w1a = wx0[:D_FEAT].astype(bf16)
    w1b = jnp.concatenate([wx0[D_FEAT:].astype(bf16),
                           jnp.zeros((128 - CE, HID), bf16)], axis=0)
    w2x = (p['W_x'][1] / jnp.sqrt(jnp.float32(HID))).astype(bf16)
    w3x = (p['W_x'][2] / jnp.sqrt(jnp.float32(HID))).astype(bf16)

    x_out = pl.pallas_call(
        _edge_mlp_body,
        grid=(NB,),
        in_specs=[
            pl.BlockSpec((BLK, D_FEAT), lambda i: (i, 0)),
            pl.BlockSpec((BLK, 128), lambda i: (i, 0)),
            pl.BlockSpec((3, BLK), lambda i: (0, i)),
            pl.BlockSpec((D_FEAT, HID), lambda i: (0, 0)),
            pl.BlockSpec((128, HID), lambda i: (0, 0)),
            pl.BlockSpec((HID, HID), lambda i: (0, 0)),
            pl.BlockSpec((HID, HID), lambda i: (0, 0)),
        ],
        out_specs=pl.BlockSpec((BLK, HID), lambda i: (i, 0)),
        out_shape=jax.ShapeDtypeStruct((N_EDGES, HID), f32),
    )(x, w_edges, vectors.T, w1a, w1b, w2x, w3x)

    charges = q_pad[:N_NODES, 0]
    pot = pot_arr[0, 0]
    return x_out, V, charges, pot


# speedup vs baseline: 2.0080x; 1.0401x over previous
import functools

import jax
import jax.numpy as jnp
from jax import lax
from jax.experimental import pallas as pl
from jax.experimental.pallas import tpu as pltpu
from jax.experimental.pallas import tpu_sc as plsc

f32 = jnp.float32
bf16 = jnp.bfloat16

N_EDGES = 160000
N_NODES = 10000
NP = 10240
D_FEAT = 256
CE = 64
HID = 512

BLK = 3200
NB = N_EDGES // BLK

SC_ROWS = 1280
SC_COLS = 125
SC_TILES = 32
SC_RPT = SC_ROWS // SC_TILES

_EA = -28.0
_EB = 48.0
_EC = -21.0


def _chi_body(x_ref, w1_ref, w2_ref, w3_ref, chi_ref):
    h = jnp.dot(x_ref[:].astype(bf16), w1_ref[:], preferred_element_type=f32)
    h = h * jax.nn.sigmoid(h)
    h = jnp.dot(h.astype(bf16), w2_ref[:], preferred_element_type=f32)
    h = h * jax.nn.sigmoid(h)
    chi_t = lax.dot_general(w3_ref[:], h.astype(bf16),
                            (((1,), (1,)), ((), ())),
                            preferred_element_type=f32)
    chi_ref[:] = chi_t.reshape(1, 1, BLK)


def _node_body(p_ref, sp_ref, tab_ref, w0a_ref, w0b_ref, w1w_ref, sc_ref,
               q_ref, w_ref, pot_ref):
    chi_scale = sc_ref[0, 0]
    gamma_scale = sc_ref[0, 1]
    gamma_shift = sc_ref[0, 2]
    chis = (p_ref[:, 0:1] + p_ref[:, 1:2]) * chi_scale
    ks = lax.broadcasted_iota(jnp.int32, (NP, 128), 1)
    oh = (sp_ref[:] == ks).astype(f32)
    g = jnp.dot(oh, tab_ref[:], preferred_element_type=f32,
                precision=jax.lax.Precision.HIGHEST)
    ce = g[:, 0:CE]
    gam = g[:, CE:CE + 1] * gamma_scale + gamma_shift
    hraw = g[:, CE + 1:CE + 2]
    soft = jnp.maximum(hraw, 0.0) + jnp.log(1.0 + jnp.exp(-jnp.abs(hraw)))
    h_eff = soft + 1.0 / gam
    q = -chis / h_eff
    q_ref[:] = q
    pot_ref[:, :] = jnp.sum(0.5 * h_eff * q * q + chis * q).reshape(1, 1)
    h = q * w0a_ref[:] + jnp.dot(ce, w0b_ref[:], preferred_element_type=f32,
                                 precision=jax.lax.Precision.HIGHEST)
    h = h * jax.nn.sigmoid(h)
    w_ref[:] = jnp.dot(h, w1w_ref[:], preferred_element_type=f32,
                       precision=jax.lax.Precision.HIGHEST)


def _edge_mlp_body(x_ref, we_ref, v_ref, w1a_ref, w1b_ref, w2_ref, w3_ref,
                   out_ref):
    HB = BLK // 4
    v = v_ref[:]
    d2 = lax.transpose(jnp.sum(v * v, axis=0, keepdims=True), (1, 0))
    d = jnp.sqrt(d2)
    u = 1.0 + _EA * d**6 + _EB * d**7 + _EC * d**8
    env = jnp.where(d < 1.0, u, 0.0)
    for k in range(4):
        a, b = k * HB, (k + 1) * HB
        h = jnp.dot(x_ref[a:b, :].astype(bf16), w1a_ref[:],
                    preferred_element_type=f32)
        h = (h + jnp.dot(we_ref[a:b, :].astype(bf16), w1b_ref[:],
                         preferred_element_type=f32)).astype(bf16)
        h = h * jax.nn.sigmoid(h)
        h = jnp.dot(h, w2_ref[:], preferred_element_type=f32).astype(bf16)
        h = h * jax.nn.sigmoid(h)
        h = jnp.dot(h, w3_ref[:], preferred_element_type=f32)
        out_ref[a:b, :] = env[a:b, :] * h


SC_EPT = N_EDGES // SC_TILES
SC_GCHUNK = 128
SC_GFULL = SC_EPT // SC_GCHUNK
SC_GTAIL = SC_EPT - SC_GFULL * SC_GCHUNK


def _sc_scatter_add(chis2d, snd2d, zeros_np):
    mesh = plsc.VectorSubcoreMesh(core_axis_name="c", subcore_axis_name="s")

    @functools.partial(
        pl.kernel,
        out_type=jax.ShapeDtypeStruct((2 * NP,), f32),
        mesh=mesh,
        scratch_types=[
            pltpu.VMEM((SC_RPT, SC_COLS), jnp.int32),
            pltpu.VMEM((SC_RPT, SC_COLS), f32),
            pltpu.VMEM_SHARED((NP,), f32),
        ],
    )
    def _scatter_k(chis_hbm, snd_hbm, zero_hbm, out_hbm, idx_v, val_v, acc_sh):
        c = lax.axis_index("c")
        s = lax.axis_index("s")
        wid = c * 16 + s

        @pl.when(s == 0)
        def _():
            pltpu.sync_copy(zero_hbm, acc_sh)

        plsc.subcore_barrier()
        pltpu.sync_copy(snd_hbm.at[pl.ds(wid * SC_RPT, SC_RPT)], idx_v)
        pltpu.sync_copy(chis_hbm.at[pl.ds(wid * SC_RPT, SC_RPT)], val_v)

        def body(j, carry):
            pltpu.sync_copy(val_v.at[j], acc_sh.at[idx_v.at[j]], add=True)
            return carry

        lax.fori_loop(0, SC_RPT, body, 0)
        plsc.subcore_barrier()

        @pl.when(s == 0)
        def _():
            pltpu.sync_copy(acc_sh, out_hbm.at[pl.ds(c * NP, NP)])

    return _scatter_k(chis2d, snd2d, zeros_np)


def _sc_gather_rows(w_nodes, snd_flat):
    mesh = plsc.VectorSubcoreMesh(core_axis_name="c", subcore_axis_name="s")

    @functools.partial(
        pl.kernel,
        out_type=jax.ShapeDtypeStruct((N_EDGES, 128), f32),
        mesh=mesh,
        scratch_types=[
            pltpu.VMEM((SC_EPT,), jnp.int32),
            pltpu.VMEM((SC_GCHUNK, 128), f32),
            pltpu.VMEM((SC_GCHUNK, 128), f32),
            pltpu.VMEM((SC_GTAIL, 128), f32),
            pltpu.SemaphoreType.DMA,
            pltpu.SemaphoreType.DMA,
        ],
    )
    def _gather_k(w_hbm, snd_hbm, out_hbm, idx_v, buf0, buf1, tail_v,
                  sem0, sem1):
        c = lax.axis_index("c")
        s = lax.axis_index("s")
        wid = c * 16 + s
        base = wid * SC_EPT
        pltpu.sync_copy(snd_hbm.at[pl.ds(base, SC_EPT)], idx_v)
        pltpu.async_copy(w_hbm.at[idx_v.at[pl.ds(0, SC_GCHUNK)]], buf0, sem0)
        npair = SC_GFULL // 2

        def body(i, carry):
            pltpu.async_copy(
                w_hbm.at[idx_v.at[pl.ds((2 * i + 1) * SC_GCHUNK, SC_GCHUNK)]],
                buf1, sem1)
            pltpu.make_async_copy(w_hbm.at[pl.ds(0, SC_GCHUNK)], buf0,
                                  sem0).wait()
            pltpu.sync_copy(
                buf0, out_hbm.at[pl.ds(base + 2 * i * SC_GCHUNK, SC_GCHUNK)])

            @pl.when(i < npair - 1)
            def _():
                pltpu.async_copy(
                    w_hbm.at[idx_v.at[pl.ds((2 * i + 2) * SC_GCHUNK,
                                            SC_GCHUNK)]],
                    buf0, sem0)

            pltpu.make_async_copy(w_hbm.at[pl.ds(0, SC_GCHUNK)], buf1,
                                  sem1).wait()
            pltpu.sync_copy(
                buf1,
                out_hbm.at[pl.ds(base + (2 * i + 1) * SC_GCHUNK, SC_GCHUNK)])
            return carry

        lax.fori_loop(0, npair, body, 0)
        pltpu.async_copy(
            w_hbm.at[idx_v.at[pl.ds((SC_GFULL - 1) * SC_GCHUNK, SC_GCHUNK)]],
            buf0, sem0)
        pltpu.async_copy(
            w_hbm.at[idx_v.at[pl.ds(SC_GFULL * SC_GCHUNK, SC_GTAIL)]],
            tail_v, sem1)
        pltpu.make_async_copy(w_hbm.at[pl.ds(0, SC_GCHUNK)], buf0,
                              sem0).wait()
        pltpu.sync_copy(
            buf0,
            out_hbm.at[pl.ds(base + (SC_GFULL - 1) * SC_GCHUNK, SC_GCHUNK)])
        pltpu.make_async_copy(w_hbm.at[pl.ds(0, SC_GTAIL)], tail_v,
                              sem1).wait()
        pltpu.sync_copy(
            tail_v,
            out_hbm.at[pl.ds(base + SC_GFULL * SC_GCHUNK, SC_GTAIL)])

    return _gather_k(w_nodes, snd_flat)


def kernel(vectors, x, V, params, senders, species):
    p = params

    w1c = (p['W_chi'][0] / jnp.sqrt(jnp.float32(D_FEAT))).astype(bf16)
    w2c = (p['W_chi'][1] / jnp.sqrt(jnp.float32(CE))).astype(bf16)
    w3c = (p['W_chi'][2][:, 0] / jnp.sqrt(jnp.float32(CE))
           ).reshape(1, CE).astype(bf16)

    chi_e = pl.pallas_call(
        _chi_body,
        grid=(NB,),
        in_specs=[
            pl.BlockSpec((BLK, D_FEAT), lambda i: (i, 0)),
            pl.BlockSpec((D_FEAT, CE), lambda i: (0, 0)),
            pl.BlockSpec((CE, CE), lambda i: (0, 0)),
            pl.BlockSpec((1, CE), lambda i: (0, 0)),
        ],
        out_specs=pl.BlockSpec((1, 1, BLK), lambda i: (i, 0, 0)),
        out_shape=jax.ShapeDtypeStruct((NB, 1, BLK), f32),
    )(x, w1c, w2c, w3c)

    chis2d = chi_e.reshape(SC_ROWS, SC_COLS)
    snd2d = senders.reshape(SC_ROWS, SC_COLS)
    partials = _sc_scatter_add(chis2d, snd2d,
                               jnp.zeros((NP,), f32)).reshape(2, NP)

    table = jnp.concatenate([
        p['charge_embed'], p['radius'][:, None], p['hardness'][:, None],
        jnp.zeros((100, 128 - CE - 2), f32)], axis=1)
    table = jnp.concatenate([table, jnp.zeros((28, 128), f32)], axis=0)
    w0 = p['W_w'][0] / jnp.sqrt(jnp.float32(1 + CE))
    w0a = w0[0:1, :]
    w0b = w0[1:, :]
    w1w = jnp.concatenate([
        p['W_w'][1] / jnp.sqrt(jnp.float32(CE)),
        jnp.zeros((CE, 128 - CE), f32)], axis=1)
    scal = jnp.stack([p['chi_scale'], p['gamma_scale'],
                      p['gamma_shift']]).reshape(1, 3)
    sp_col = jnp.pad(species, (0, NP - N_NODES)).reshape(NP, 1)

    q_pad, w_nodes, pot_arr = pl.pallas_call(
        _node_body,
        grid=(1,),
        in_specs=[
            pl.BlockSpec((NP, 2), lambda i: (0, 0)),
            pl.BlockSpec((NP, 1), lambda i: (0, 0)),
            pl.BlockSpec((128, 128), lambda i: (0, 0)),
            pl.BlockSpec((1, CE), lambda i: (0, 0)),
            pl.BlockSpec((CE, CE), lambda i: (0, 0)),
            pl.BlockSpec((CE, 128), lambda i: (0, 0)),
            pl.BlockSpec((1, 3), lambda i: (0, 0)),
        ],
        out_specs=[
            pl.BlockSpec((NP, 1), lambda i: (0, 0)),
            pl.BlockSpec((NP, 128), lambda i: (0, 0)),
            pl.BlockSpec((1, 1), lambda i: (0, 0)),
        ],
        out_shape=[
            jax.ShapeDtypeStruct((NP, 1), f32),
            jax.ShapeDtypeStruct((NP, 128), f32),
            jax.ShapeDtypeStruct((1, 1), f32),
        ],
    )(partials.T, sp_col, table, w0a, w0b, w1w, scal)

    w_edges = _sc_gather_rows(w_nodes, senders)

    wx0 = p['W_x'][0] / jnp.sqrt(jnp.float32(D_FEAT + CE))
    w1a = wx0[:D_FEAT].astype(bf16)
    w1b = jnp.concatenate([wx0[D_FEAT:].astype(bf16),
                           jnp.zeros((128 - CE, HID), bf16)], axis=0)
    w2x = (p['W_x'][1] / jnp.sqrt(jnp.float32(HID))).astype(bf16)
    w3x = (p['W_x'][2] / jnp.sqrt(jnp.float32(HID))).astype(bf16)

    x_out = pl.pallas_call(
        _edge_mlp_body,
        grid=(NB,),
        in_specs=[
            pl.BlockSpec((BLK, D_FEAT), lambda i: (i, 0)),
            pl.BlockSpec((BLK, 128), lambda i: (i, 0)),
            pl.BlockSpec((3, BLK), lambda i: (0, i)),
            pl.BlockSpec((D_FEAT, HID), lambda i: (0, 0)),
            pl.BlockSpec((128, HID), lambda i: (0, 0)),
            pl.BlockSpec((HID, HID), lambda i: (0, 0)),
            pl.BlockSpec((HID, HID), lambda i: (0, 0)),
        ],
        out_specs=pl.BlockSpec((BLK, HID), lambda i: (i, 0)),
        out_shape=jax.ShapeDtypeStruct((N_EDGES, HID), f32),
    )(x, w_edges, vectors.T, w1a, w1b, w2x, w3x)

    charges = q_pad[:N_NODES, 0]
    pot = pot_arr[0, 0]
    return x_out, V, charges, pot
